# baseline (speedup 1.0000x reference)
import functools

import jax
import jax.numpy as jnp
from jax import lax
from jax.experimental import pallas as pl
from jax.experimental.pallas import tpu as pltpu
from jax.experimental.pallas import tpu_sc as plsc

NC, NS, L = 2, 16, 16
NW = NC * NS
B = 16384
D = 64
NE = 1_000_000
NR = 1000
BPW = B // NW
G = D // L
RPB = 16
NBLK = BPW // RPB

TC = 2048


def _transpose_body(t_ref, o_ref):
    o_ref[...] = t_ref[...].T


def _to_row_major(table_t, n):
    grid = (n + TC - 1) // TC
    return pl.pallas_call(
        _transpose_body,
        grid=(grid,),
        in_specs=[pl.BlockSpec((D, TC), lambda c: (0, c))],
        out_specs=pl.BlockSpec((TC, D), lambda c: (c, 0)),
        out_shape=jax.ShapeDtypeStruct((n, D), jnp.float32),
    )(table_t)


_mesh = plsc.VectorSubcoreMesh(core_axis_name="c", subcore_axis_name="s")


@functools.partial(
    pl.kernel,
    out_type=jax.ShapeDtypeStruct((B,), jnp.float32),
    mesh=_mesh,
    scratch_types=[
        pltpu.VMEM((BPW,), jnp.int32),
        pltpu.VMEM((BPW,), jnp.int32),
        pltpu.VMEM((BPW,), jnp.int32),
        pltpu.VMEM((BPW // 2, 2 * D), jnp.float32),
        pltpu.VMEM((BPW // 2, 2 * D), jnp.float32),
        pltpu.VMEM((BPW // 2, 2 * D), jnp.float32),
        pltpu.VMEM((BPW,), jnp.float32),
        pltpu.SemaphoreType.DMA,
    ],
)
def _transe(head_h, rel_h, tail_h, ent_h, relemb_h, out_h,
            ih_v, ir_v, it_v, h_v, r_v, t_v, o_v, sem):
    wid = lax.axis_index("s") * NC + lax.axis_index("c")

    pltpu.sync_copy(head_h.at[wid], ih_v)
    pltpu.sync_copy(rel_h.at[wid], ir_v)
    pltpu.sync_copy(tail_h.at[wid], it_v)

    def fire(g, c):
        base = g * L
        ihv = ih_v[pl.ds(base, L)]
        irv = ir_v[pl.ds(base, L)]
        itv = it_v[pl.ds(base, L)]
        for rr in range(L):
            i = base + rr
            half = pl.ds((rr % 2) * D, D)
            pltpu.async_copy(ent_h.at[ihv[rr]], h_v.at[i // 2, half], sem)
            pltpu.async_copy(relemb_h.at[irv[rr]], r_v.at[i // 2, half], sem)
            pltpu.async_copy(ent_h.at[itv[rr]], t_v.at[i // 2, half], sem)
        return c

    lax.fori_loop(0, BPW // L, fire, 0)
    pltpu.make_async_copy(ent_h.at[pl.ds(0, BPW)], h_v, sem).wait()
    pltpu.make_async_copy(ent_h.at[pl.ds(0, BPW)], r_v, sem).wait()
    pltpu.make_async_copy(ent_h.at[pl.ds(0, BPW)], t_v, sem).wait()

    iot = lax.iota(jnp.int32, L)
    _dnums = lax.GatherDimensionNumbers(
        offset_dims=(), collapsed_slice_dims=(0,), start_index_map=(0,))

    def _perm(v, idx):
        return lax.gather(v, idx.reshape(L, 1), _dnums, (1,),
                          mode=lax.GatherScatterMode.PROMISE_IN_BOUNDS)

    def _hsum(v):
        for s in (8, 4, 2, 1):
            v = v + _perm(v, jnp.bitwise_xor(iot, s))
        return v

    def blk_body(blk, carry):
        rbase = blk * RPB
        outv = jnp.zeros((L,), jnp.float32)
        for rr in range(RPB):
            row = rbase + rr
            acc = jnp.zeros((L,), jnp.float32)
            for g in range(G):
                sl = pl.ds((rr % 2) * D + g * L, L)
                acc = acc + jnp.abs(h_v[row // 2, sl] + r_v[row // 2, sl]
                                    - t_v[row // 2, sl])
            outv = jnp.where(iot == rr, _hsum(acc), outv)
        o_v[pl.ds(rbase, RPB)] = -outv
        return carry

    lax.fori_loop(0, NBLK, blk_body, 0)
    pltpu.sync_copy(o_v, out_h.at[pl.ds(wid * BPW, BPW)])


def kernel(head, relation, tail, entity_emb, relation_emb):
    head2 = head.astype(jnp.int32).reshape(NW, BPW)
    rel2 = relation.astype(jnp.int32).reshape(NW, BPW)
    tail2 = tail.astype(jnp.int32).reshape(NW, BPW)
    ent_rm = _to_row_major(entity_emb.T, NE)
    rel_rm = _to_row_major(relation_emb.T, NR)
    return _transe(head2, rel2, tail2, ent_rm, rel_rm)


# speedup vs baseline: 1.6486x; 1.6486x over previous
import functools

import jax
import jax.numpy as jnp
from jax import lax
from jax.experimental import pallas as pl
from jax.experimental.pallas import tpu as pltpu
from jax.experimental.pallas import tpu_sc as plsc

NC, NS, L = 2, 16, 16
NW = NC * NS
B = 16384
D = 64
NE = 1_000_000
NR = 1000
BPW = B // NW
G = D // L
RPB = 16
NBLK = BPW // RPB

TC = 8192


def _transpose_body(t_ref, o_ref):
    o_ref[...] = t_ref[...].T


def _to_row_major(table_t, n):
    grid = (n + TC - 1) // TC
    return pl.pallas_call(
        _transpose_body,
        grid=(grid,),
        in_specs=[pl.BlockSpec((D, TC), lambda c: (0, c))],
        out_specs=pl.BlockSpec((TC, D), lambda c: (c, 0)),
        out_shape=jax.ShapeDtypeStruct((n, D), jnp.float32),
    )(table_t)


_mesh = plsc.VectorSubcoreMesh(core_axis_name="c", subcore_axis_name="s")


@functools.partial(
    pl.kernel,
    out_type=jax.ShapeDtypeStruct((B,), jnp.float32),
    mesh=_mesh,
    scratch_types=[
        pltpu.VMEM((BPW,), jnp.int32),
        pltpu.VMEM((BPW,), jnp.int32),
        pltpu.VMEM((BPW,), jnp.int32),
        pltpu.VMEM((BPW // 2, 2 * D), jnp.float32),
        pltpu.VMEM((BPW // 2, 2 * D), jnp.float32),
        pltpu.VMEM((BPW // 2, 2 * D), jnp.float32),
        pltpu.VMEM((BPW,), jnp.float32),
        pltpu.SemaphoreType.DMA,
    ],
)
def _transe(head_h, rel_h, tail_h, ent_h, relemb_h, out_h,
            ih_v, ir_v, it_v, h_v, r_v, t_v, o_v, sem):
    wid = lax.axis_index("s") * NC + lax.axis_index("c")

    pltpu.sync_copy(head_h.at[wid], ih_v)
    pltpu.sync_copy(rel_h.at[wid], ir_v)
    pltpu.sync_copy(tail_h.at[wid], it_v)

    def fire(g, c):
        base = g * L
        ihv = ih_v[pl.ds(base, L)]
        irv = ir_v[pl.ds(base, L)]
        itv = it_v[pl.ds(base, L)]
        for rr in range(L):
            i = base + rr
            half = pl.ds((rr % 2) * D, D)
            pltpu.async_copy(ent_h.at[ihv[rr]], h_v.at[i // 2, half], sem)
            pltpu.async_copy(relemb_h.at[irv[rr]], r_v.at[i // 2, half], sem)
            pltpu.async_copy(ent_h.at[itv[rr]], t_v.at[i // 2, half], sem)
        return c

    lax.fori_loop(0, BPW // L, fire, 0)
    pltpu.make_async_copy(ent_h.at[pl.ds(0, BPW)], h_v, sem).wait()
    pltpu.make_async_copy(ent_h.at[pl.ds(0, BPW)], r_v, sem).wait()
    pltpu.make_async_copy(ent_h.at[pl.ds(0, BPW)], t_v, sem).wait()

    iot = lax.iota(jnp.int32, L)
    _dnums = lax.GatherDimensionNumbers(
        offset_dims=(), collapsed_slice_dims=(0,), start_index_map=(0,))

    def _perm(v, idx):
        return lax.gather(v, idx.reshape(L, 1), _dnums, (1,),
                          mode=lax.GatherScatterMode.PROMISE_IN_BOUNDS)

    def _hsum(v):
        for s in (8, 4, 2, 1):
            v = v + _perm(v, jnp.bitwise_xor(iot, s))
        return v

    def blk_body(blk, carry):
        rbase = blk * RPB
        outv = jnp.zeros((L,), jnp.float32)
        for rr in range(RPB):
            row = rbase + rr
            acc = jnp.zeros((L,), jnp.float32)
            for g in range(G):
                sl = pl.ds((rr % 2) * D + g * L, L)
                acc = acc + jnp.abs(h_v[row // 2, sl] + r_v[row // 2, sl]
                                    - t_v[row // 2, sl])
            outv = jnp.where(iot == rr, _hsum(acc), outv)
        o_v[pl.ds(rbase, RPB)] = -outv
        return carry

    lax.fori_loop(0, NBLK, blk_body, 0)
    pltpu.sync_copy(o_v, out_h.at[pl.ds(wid * BPW, BPW)])


def kernel(head, relation, tail, entity_emb, relation_emb):
    head2 = head.astype(jnp.int32).reshape(NW, BPW)
    rel2 = relation.astype(jnp.int32).reshape(NW, BPW)
    tail2 = tail.astype(jnp.int32).reshape(NW, BPW)
    ent_rm = _to_row_major(entity_emb.T, NE)
    rel_rm = _to_row_major(relation_emb.T, NR)
    return _transe(head2, rel2, tail2, ent_rm, rel_rm)


# speedup vs baseline: 1.7884x; 1.0848x over previous
import functools

import jax
import jax.numpy as jnp
from jax import lax
from jax.experimental import pallas as pl
from jax.experimental.pallas import tpu as pltpu
from jax.experimental.pallas import tpu_sc as plsc

NC, NS, L = 2, 16, 16
NW = NC * NS
B = 16384
D = 64
NE = 1_000_000
NR = 1000
BPW = B // NW
G = D // L
RPB = 16
NBLK = BPW // RPB

TC = 32768


def _transpose_body(t_ref, o_ref):
    o_ref[...] = t_ref[...].T


def _to_row_major(table_t, n):
    grid = (n + TC - 1) // TC
    return pl.pallas_call(
        _transpose_body,
        grid=(grid,),
        in_specs=[pl.BlockSpec((D, TC), lambda c: (0, c))],
        out_specs=pl.BlockSpec((TC, D), lambda c: (c, 0)),
        out_shape=jax.ShapeDtypeStruct((n, D), jnp.float32),
    )(table_t)


_mesh = plsc.VectorSubcoreMesh(core_axis_name="c", subcore_axis_name="s")


@functools.partial(
    pl.kernel,
    out_type=jax.ShapeDtypeStruct((B,), jnp.float32),
    mesh=_mesh,
    scratch_types=[
        pltpu.VMEM((BPW,), jnp.int32),
        pltpu.VMEM((BPW,), jnp.int32),
        pltpu.VMEM((BPW,), jnp.int32),
        pltpu.VMEM((BPW // 2, 2 * D), jnp.float32),
        pltpu.VMEM((BPW // 2, 2 * D), jnp.float32),
        pltpu.VMEM((BPW // 2, 2 * D), jnp.float32),
        pltpu.VMEM((BPW,), jnp.float32),
        pltpu.SemaphoreType.DMA,
    ],
)
def _transe(head_h, rel_h, tail_h, ent_h, relemb_h, out_h,
            ih_v, ir_v, it_v, h_v, r_v, t_v, o_v, sem):
    wid = lax.axis_index("s") * NC + lax.axis_index("c")

    pltpu.sync_copy(head_h.at[wid], ih_v)
    pltpu.sync_copy(rel_h.at[wid], ir_v)
    pltpu.sync_copy(tail_h.at[wid], it_v)

    def fire(g, c):
        base = g * L
        ihv = ih_v[pl.ds(base, L)]
        irv = ir_v[pl.ds(base, L)]
        itv = it_v[pl.ds(base, L)]
        for rr in range(L):
            i = base + rr
            half = pl.ds((rr % 2) * D, D)
            pltpu.async_copy(ent_h.at[ihv[rr]], h_v.at[i // 2, half], sem)
            pltpu.async_copy(relemb_h.at[irv[rr]], r_v.at[i // 2, half], sem)
            pltpu.async_copy(ent_h.at[itv[rr]], t_v.at[i // 2, half], sem)
        return c

    lax.fori_loop(0, BPW // L, fire, 0)
    pltpu.make_async_copy(ent_h.at[pl.ds(0, BPW)], h_v, sem).wait()
    pltpu.make_async_copy(ent_h.at[pl.ds(0, BPW)], r_v, sem).wait()
    pltpu.make_async_copy(ent_h.at[pl.ds(0, BPW)], t_v, sem).wait()

    iot = lax.iota(jnp.int32, L)
    _dnums = lax.GatherDimensionNumbers(
        offset_dims=(), collapsed_slice_dims=(0,), start_index_map=(0,))

    def _perm(v, idx):
        return lax.gather(v, idx.reshape(L, 1), _dnums, (1,),
                          mode=lax.GatherScatterMode.PROMISE_IN_BOUNDS)

    def _hsum(v):
        for s in (8, 4, 2, 1):
            v = v + _perm(v, jnp.bitwise_xor(iot, s))
        return v

    def blk_body(blk, carry):
        rbase = blk * RPB
        outv = jnp.zeros((L,), jnp.float32)
        for rr in range(RPB):
            row = rbase + rr
            acc = jnp.zeros((L,), jnp.float32)
            for g in range(G):
                sl = pl.ds((rr % 2) * D + g * L, L)
                acc = acc + jnp.abs(h_v[row // 2, sl] + r_v[row // 2, sl]
                                    - t_v[row // 2, sl])
            outv = jnp.where(iot == rr, _hsum(acc), outv)
        o_v[pl.ds(rbase, RPB)] = -outv
        return carry

    lax.fori_loop(0, NBLK, blk_body, 0)
    pltpu.sync_copy(o_v, out_h.at[pl.ds(wid * BPW, BPW)])


def kernel(head, relation, tail, entity_emb, relation_emb):
    head2 = head.astype(jnp.int32).reshape(NW, BPW)
    rel2 = relation.astype(jnp.int32).reshape(NW, BPW)
    tail2 = tail.astype(jnp.int32).reshape(NW, BPW)
    ent_rm = _to_row_major(entity_emb.T, NE)
    rel_rm = _to_row_major(relation_emb.T, NR)
    return _transe(head2, rel2, tail2, ent_rm, rel_rm)


# speedup vs baseline: 1.9905x; 1.1130x over previous
import functools

import jax
import jax.numpy as jnp
from jax import lax
from jax.experimental import pallas as pl
from jax.experimental.pallas import tpu as pltpu
from jax.experimental.pallas import tpu_sc as plsc

NC, NS, L = 2, 16, 16
NW = NC * NS
B = 16384
D = 64
NE = 1_000_000
NR = 1000
BPW = B // NW
G = D // L
RPB = 16
CHK = 256
NPASS = BPW // CHK


def _make_packer(tcw):
    hs = tcw // 2

    def _body(t_ref, o_ref):
        x = t_ref[...]
        o_ref[:, 0:D] = x[:, 0:hs].T
        o_ref[:, D:2 * D] = x[:, hs:tcw].T
    return _body


def _to_packed(table_t, n, tcw):
    grid = (n + tcw - 1) // tcw
    return pl.pallas_call(
        _make_packer(tcw),
        grid=(grid,),
        in_specs=[pl.BlockSpec((D, tcw), lambda c: (0, c))],
        out_specs=pl.BlockSpec((tcw // 2, 2 * D), lambda c: (c, 0)),
        out_shape=jax.ShapeDtypeStruct((grid * (tcw // 2), 2 * D), jnp.float32),
    )(table_t)


TCW_E = 32768
TCW_R = 1024
SH_E, SH_R = 14, 9


def _pack_row(v, sh):
    return jnp.bitwise_or(
        lax.shift_left(lax.shift_right_logical(v, sh + 1), sh),
        jnp.bitwise_and(v, (1 << sh) - 1))


def _pack_off(v, sh):
    return jnp.bitwise_and(lax.shift_right_logical(v, sh), 1) * D


_mesh = plsc.VectorSubcoreMesh(core_axis_name="c", subcore_axis_name="s")


@functools.partial(
    pl.kernel,
    out_type=jax.ShapeDtypeStruct((B,), jnp.float32),
    mesh=_mesh,
    scratch_types=[
        pltpu.VMEM((BPW,), jnp.int32),
        pltpu.VMEM((BPW,), jnp.int32),
        pltpu.VMEM((BPW,), jnp.int32),
        pltpu.VMEM((CHK, 2 * D), jnp.float32),
        pltpu.VMEM((CHK, 2 * D), jnp.float32),
        pltpu.VMEM((CHK, 2 * D), jnp.float32),
        pltpu.VMEM((BPW,), jnp.float32),
        pltpu.SemaphoreType.DMA,
    ],
)
def _transe(head_h, rel_h, tail_h, ent_h, relemb_h, out_h,
            ih_v, ir_v, it_v, h_v, r_v, t_v, o_v, sem):
    wid = lax.axis_index("s") * NC + lax.axis_index("c")

    pltpu.sync_copy(head_h.at[wid], ih_v)
    pltpu.sync_copy(rel_h.at[wid], ir_v)
    pltpu.sync_copy(tail_h.at[wid], it_v)

    iot = lax.iota(jnp.int32, L)
    _dnums = lax.GatherDimensionNumbers(
        offset_dims=(), collapsed_slice_dims=(0,), start_index_map=(0,))

    def _perm(v, idx):
        return lax.gather(v, idx.reshape(L, 1), _dnums, (1,),
                          mode=lax.GatherScatterMode.PROMISE_IN_BOUNDS)

    def _hsum(v):
        for s in (8, 4, 2, 1):
            v = v + _perm(v, jnp.bitwise_xor(iot, s))
        return v

    for p in range(NPASS):
        pbase = p * CHK

        def fire(g, c, pbase=pbase):
            base = pbase + g * L
            ihv = ih_v[pl.ds(base, L)]
            irv = ir_v[pl.ds(base, L)]
            itv = it_v[pl.ds(base, L)]
            ihr = _pack_row(ihv, SH_E)
            irr = _pack_row(irv, SH_R)
            itr = _pack_row(itv, SH_E)
            for rr in range(L):
                i = g * L + rr
                pltpu.async_copy(ent_h.at[ihr[rr]], h_v.at[i], sem)
                pltpu.async_copy(relemb_h.at[irr[rr]], r_v.at[i], sem)
                pltpu.async_copy(ent_h.at[itr[rr]], t_v.at[i], sem)
            return c

        lax.fori_loop(0, CHK // L, fire, 0)
        pltpu.make_async_copy(ent_h.at[pl.ds(0, CHK)], h_v, sem).wait()
        pltpu.make_async_copy(ent_h.at[pl.ds(0, CHK)], r_v, sem).wait()
        pltpu.make_async_copy(ent_h.at[pl.ds(0, CHK)], t_v, sem).wait()

        def blk_body(blk, carry, pbase=pbase):
            rbase = blk * RPB
            ihv = ih_v[pl.ds(pbase + rbase, L)]
            irv = ir_v[pl.ds(pbase + rbase, L)]
            itv = it_v[pl.ds(pbase + rbase, L)]
            oh = _pack_off(ihv, SH_E)
            orr = _pack_off(irv, SH_R)
            ot = _pack_off(itv, SH_E)
            outv = jnp.zeros((L,), jnp.float32)
            for rr in range(RPB):
                row = rbase + rr
                ph, pr, pt = oh[rr], orr[rr], ot[rr]
                acc = jnp.zeros((L,), jnp.float32)
                for g in range(G):
                    acc = acc + jnp.abs(
                        h_v[row, pl.ds(ph + g * L, L)]
                        + r_v[row, pl.ds(pr + g * L, L)]
                        - t_v[row, pl.ds(pt + g * L, L)])
                outv = jnp.where(iot == rr, _hsum(acc), outv)
            o_v[pl.ds(pbase + rbase, RPB)] = -outv
            return carry

        lax.fori_loop(0, CHK // RPB, blk_body, 0)

    pltpu.sync_copy(o_v, out_h.at[pl.ds(wid * BPW, BPW)])


def kernel(head, relation, tail, entity_emb, relation_emb):
    head2 = head.astype(jnp.int32).reshape(NW, BPW)
    rel2 = relation.astype(jnp.int32).reshape(NW, BPW)
    tail2 = tail.astype(jnp.int32).reshape(NW, BPW)
    ent_pk = _to_packed(entity_emb.T, NE, TCW_E)
    rel_pk = _to_packed(relation_emb.T, NR, TCW_R)
    return _transe(head2, rel2, tail2, ent_pk, rel_pk)
